# Initial kernel scaffold; baseline (speedup 1.0000x reference)
import jax
import jax.numpy as jnp
from jax.experimental import pallas as pl


def kernel(input_ids, token_type_ids, position_ids, word_emb, pos_emb, type_emb):
    raise NotImplementedError("write your pallas kernel here")


# speedup vs baseline: 3.5732x; 3.5732x over previous
import functools

import jax
import jax.numpy as jnp
from jax import lax
from jax.experimental import pallas as pl
from jax.experimental.pallas import tpu as pltpu
from jax.experimental.pallas import tpu_sc as plsc

EMBED = 128
MAX_POS = 4096
TYPE_VOCAB = 2

NC, NS, LANES = 2, 16, 16
NW = NC * NS


def _combined_body(pos_ref, type_ref, out_ref):
    t = pl.program_id(0)
    rows = type_ref[...]
    row = jnp.where(t == 0, rows[0:1, :], rows[1:2, :])
    out_ref[...] = pos_ref[...] + row


def _build_combined(pos_emb, type_emb):
    return pl.pallas_call(
        _combined_body,
        grid=(TYPE_VOCAB,),
        in_specs=[
            pl.BlockSpec((MAX_POS, EMBED), lambda t: (0, 0)),
            pl.BlockSpec((TYPE_VOCAB, EMBED), lambda t: (0, 0)),
        ],
        out_specs=pl.BlockSpec((MAX_POS, EMBED), lambda t: (t, 0)),
        out_shape=jax.ShapeDtypeStruct((TYPE_VOCAB * MAX_POS, EMBED), jnp.float32),
    )(pos_emb, type_emb)


def _gather_sum(word_emb, comb_table, word_ids, comb_ids):
    n = word_ids.shape[0]
    assert n % NW == 0
    b_per_w = n // NW
    chunk = 256
    assert b_per_w % chunk == 0
    n_chunks = b_per_w // chunk
    mesh = plsc.VectorSubcoreMesh(core_axis_name="c", subcore_axis_name="s")

    @functools.partial(
        pl.kernel,
        mesh=mesh,
        out_type=jax.ShapeDtypeStruct((n, EMBED), jnp.float32),
        scratch_types=[
            pltpu.VMEM((chunk,), jnp.int32),
            pltpu.VMEM((chunk,), jnp.int32),
            pltpu.VMEM((chunk, EMBED), jnp.float32),
            pltpu.VMEM((chunk, EMBED), jnp.float32),
            pltpu.SemaphoreType.DMA,
            pltpu.SemaphoreType.DMA,
        ],
    )
    def k(word_hbm, comb_hbm, wid_hbm, cid_hbm, out_hbm, wi_v, ci_v, a_v, b_v, sem_a, sem_b):
        wid = lax.axis_index("c") * NS + lax.axis_index("s")
        base = wid * b_per_w

        @pl.loop(0, n_chunks)
        def _(c):
            off = base + c * chunk
            pltpu.sync_copy(wid_hbm.at[pl.ds(off, chunk)], wi_v)
            pltpu.sync_copy(cid_hbm.at[pl.ds(off, chunk)], ci_v)
            cp_a = pltpu.async_copy(word_hbm.at[wi_v], a_v, sem_a)
            cp_b = pltpu.async_copy(comb_hbm.at[ci_v], b_v, sem_b)
            cp_a.wait()
            cp_b.wait()

            @pl.loop(0, chunk)
            def _(r):
                for j in range(EMBED // LANES):
                    s = pl.ds(j * LANES, LANES)
                    a_v[r, s] = a_v[r, s] + b_v[r, s]

            pltpu.sync_copy(a_v, out_hbm.at[pl.ds(off, chunk)])

    return k(word_emb, comb_table, word_ids, comb_ids)


def kernel(input_ids, token_type_ids, position_ids, word_emb, pos_emb, type_emb):
    B, S = input_ids.shape
    n = B * S
    comb_table = _build_combined(pos_emb, type_emb)
    wid = input_ids.astype(jnp.int32).reshape(n)
    cid = (token_type_ids.astype(jnp.int32) * MAX_POS
           + position_ids.astype(jnp.int32)).reshape(n)
    out = _gather_sum(word_emb, comb_table, wid, cid)
    return out.reshape(B, S, EMBED)


# speedup vs baseline: 3.8483x; 1.0770x over previous
import functools

import jax
import jax.numpy as jnp
from jax import lax
from jax.experimental import pallas as pl
from jax.experimental.pallas import tpu as pltpu
from jax.experimental.pallas import tpu_sc as plsc

EMBED = 128
MAX_POS = 4096
TYPE_VOCAB = 2

NC, NS, LANES = 2, 16, 16
NW = NC * NS
CH = 128
ROW_UNROLL = 4


def _combined_body(pos_ref, type_ref, out_ref):
    t = pl.program_id(0)
    rows = type_ref[...]
    row = jnp.where(t == 0, rows[0:1, :], rows[1:2, :])
    out_ref[...] = pos_ref[...] + row


def _build_combined(pos_emb, type_emb):
    return pl.pallas_call(
        _combined_body,
        grid=(TYPE_VOCAB,),
        in_specs=[
            pl.BlockSpec((MAX_POS, EMBED), lambda t: (0, 0)),
            pl.BlockSpec((TYPE_VOCAB, EMBED), lambda t: (0, 0)),
        ],
        out_specs=pl.BlockSpec((MAX_POS, EMBED), lambda t: (t, 0)),
        out_shape=jax.ShapeDtypeStruct((TYPE_VOCAB * MAX_POS, EMBED), jnp.float32),
    )(pos_emb, type_emb)


def _gather_sum(word_emb, comb_table, word_ids, comb_ids, n):
    n_chunks = word_ids.shape[1]
    mesh = plsc.VectorSubcoreMesh(core_axis_name="c", subcore_axis_name="s")

    @functools.partial(
        pl.kernel,
        mesh=mesh,
        out_type=jax.ShapeDtypeStruct((n, EMBED), jnp.float32),
        scratch_types=[
            pltpu.VMEM((n_chunks, CH), jnp.int32),
            pltpu.VMEM((n_chunks, CH), jnp.int32),
            pltpu.VMEM((CH, EMBED), jnp.float32),
            pltpu.VMEM((CH, EMBED), jnp.float32),
            pltpu.VMEM((CH, EMBED), jnp.float32),
            pltpu.VMEM((CH, EMBED), jnp.float32),
            pltpu.SemaphoreType.DMA,
            pltpu.SemaphoreType.DMA,
            pltpu.SemaphoreType.DMA,
            pltpu.SemaphoreType.DMA,
            pltpu.SemaphoreType.DMA,
            pltpu.SemaphoreType.DMA,
        ],
    )
    def k(word_hbm, comb_hbm, wid_hbm, cid_hbm, out_hbm,
          wi_v, ci_v, a0, a1, b0, b1, ga0, ga1, gb0, gb1, so0, so1):
        wid = lax.axis_index("c") * NS + lax.axis_index("s")
        base = wid * (n_chunks * CH)
        a = (a0, a1)
        b = (b0, b1)
        ga = (ga0, ga1)
        gb = (gb0, gb1)
        so = (so0, so1)

        cp_wi = pltpu.async_copy(wid_hbm.at[wid], wi_v, ga0)
        cp_ci = pltpu.async_copy(cid_hbm.at[wid], ci_v, gb0)
        cp_wi.wait()
        cp_ci.wait()

        def start_gathers(c):
            p = c % 2
            cpa = pltpu.async_copy(word_hbm.at[wi_v.at[c]], a[p], ga[p])
            cpb = pltpu.async_copy(comb_hbm.at[ci_v.at[c]], b[p], gb[p])
            return cpa, cpb

        pend = {0: start_gathers(0)}
        out_pend = {}
        for c in range(n_chunks):
            p = c % 2
            if c + 1 < n_chunks:
                if c - 1 >= 0:
                    out_pend.pop(c - 1).wait()
                pend[c + 1] = start_gathers(c + 1)
            cpa, cpb = pend.pop(c)
            cpa.wait()
            cpb.wait()

            av, bv = a[p], b[p]

            @pl.loop(0, CH, step=ROW_UNROLL)
            def _(r):
                for rr in range(ROW_UNROLL):
                    for j in range(EMBED // LANES):
                        s = pl.ds(j * LANES, LANES)
                        av[r + rr, s] = av[r + rr, s] + bv[r + rr, s]

            out_pend[c] = pltpu.async_copy(
                av, out_hbm.at[pl.ds(base + c * CH, CH)], so[p])
        for c in sorted(out_pend):
            out_pend.pop(c).wait()

    return k(word_emb, comb_table, word_ids, comb_ids)


def kernel(input_ids, token_type_ids, position_ids, word_emb, pos_emb, type_emb):
    B, S = input_ids.shape
    n = B * S
    n_chunks = n // (NW * CH)
    comb_table = _build_combined(pos_emb, type_emb)
    wid = input_ids.astype(jnp.int32).reshape(NW, n_chunks, CH)
    cid = (token_type_ids.astype(jnp.int32) * MAX_POS
           + position_ids.astype(jnp.int32)).reshape(NW, n_chunks, CH)
    out = _gather_sum(word_emb, comb_table, wid, cid, n)
    return out.reshape(B, S, EMBED)


# speedup vs baseline: 4.1955x; 1.0902x over previous
import functools

import jax
import jax.numpy as jnp
from jax import lax
from jax.experimental import pallas as pl
from jax.experimental.pallas import tpu as pltpu
from jax.experimental.pallas import tpu_sc as plsc

EMBED = 128
MAX_POS = 4096
TYPE_VOCAB = 2

NC, NS, LANES = 2, 16, 16
NW = NC * NS
CH = 128
ROW_UNROLL = 4


def _combined_body(pos_ref, type_ref, out_ref):
    t = pl.program_id(0)
    rows = type_ref[...]
    row = jnp.where(t == 0, rows[0:1, :], rows[1:2, :])
    out_ref[...] = pos_ref[...] + row


def _build_combined(pos_emb, type_emb):
    return pl.pallas_call(
        _combined_body,
        grid=(TYPE_VOCAB,),
        in_specs=[
            pl.BlockSpec((MAX_POS, EMBED), lambda t: (0, 0)),
            pl.BlockSpec((TYPE_VOCAB, EMBED), lambda t: (0, 0)),
        ],
        out_specs=pl.BlockSpec((MAX_POS, EMBED), lambda t: (t, 0)),
        out_shape=jax.ShapeDtypeStruct((TYPE_VOCAB * MAX_POS, EMBED), jnp.float32),
    )(pos_emb, type_emb)


def _gather_sum(word_emb, comb_table, word_ids, comb_ids):
    B, S = word_ids.shape
    n = B * S
    b_per_w = n // NW
    n_chunks = b_per_w // CH
    w_per_row = S // b_per_w
    mesh = plsc.VectorSubcoreMesh(core_axis_name="c", subcore_axis_name="s")

    @functools.partial(
        pl.kernel,
        mesh=mesh,
        out_type=jax.ShapeDtypeStruct((n, EMBED), jnp.float32),
        scratch_types=[
            pltpu.VMEM((CH,), jnp.int32),
            pltpu.VMEM((CH,), jnp.int32),
            pltpu.VMEM((CH,), jnp.int32),
            pltpu.VMEM((CH,), jnp.int32),
            pltpu.VMEM((CH, EMBED), jnp.float32),
            pltpu.VMEM((CH, EMBED), jnp.float32),
            pltpu.VMEM((CH, EMBED), jnp.float32),
            pltpu.VMEM((CH, EMBED), jnp.float32),
            pltpu.SemaphoreType.DMA,
            pltpu.SemaphoreType.DMA,
            pltpu.SemaphoreType.DMA,
            pltpu.SemaphoreType.DMA,
            pltpu.SemaphoreType.DMA,
            pltpu.SemaphoreType.DMA,
            pltpu.SemaphoreType.DMA,
            pltpu.SemaphoreType.DMA,
        ],
    )
    def k(word_hbm, comb_hbm, wid_hbm, cid_hbm, out_hbm,
          wi0, wi1, ci0, ci1, a0, a1, b0, b1,
          si0, si1, ga0, ga1, gb0, gb1, so0, so1):
        wid = lax.axis_index("c") * NS + lax.axis_index("s")
        base = wid * b_per_w
        row = wid // w_per_row
        col0 = (wid % w_per_row) * b_per_w
        wi = (wi0, wi1)
        ci = (ci0, ci1)
        a = (a0, a1)
        b = (b0, b1)
        si = (si0, si1)
        ga = (ga0, ga1)
        gb = (gb0, gb1)
        so = (so0, so1)

        def start_ids(c):
            p = c % 2
            c1 = pltpu.async_copy(
                wid_hbm.at[row, pl.ds(col0 + c * CH, CH)], wi[p], si[p])
            c2 = pltpu.async_copy(
                cid_hbm.at[row, pl.ds(col0 + c * CH, CH)], ci[p], si[p])
            return c1, c2

        def start_gathers(c):
            p = c % 2
            cpa = pltpu.async_copy(word_hbm.at[wi[p]], a[p], ga[p])
            cpb = pltpu.async_copy(comb_hbm.at[ci[p]], b[p], gb[p])
            return cpa, cpb

        ids_pend = {0: start_ids(0)}
        for h in ids_pend.pop(0):
            h.wait()
        gat_pend = {0: start_gathers(0)}
        ids_pend[1] = start_ids(1)
        out_pend = {}

        for c in range(n_chunks):
            p = c % 2
            if c + 1 < n_chunks:
                for h in ids_pend.pop(c + 1):
                    h.wait()
                if c - 1 >= 0:
                    out_pend.pop(c - 1).wait()
                gat_pend[c + 1] = start_gathers(c + 1)
            cpa, cpb = gat_pend.pop(c)
            cpa.wait()
            cpb.wait()
            if c + 2 < n_chunks:
                ids_pend[c + 2] = start_ids(c + 2)

            av, bv = a[p], b[p]

            @pl.loop(0, CH, step=ROW_UNROLL)
            def _(r):
                for rr in range(ROW_UNROLL):
                    for j in range(EMBED // LANES):
                        s = pl.ds(j * LANES, LANES)
                        av[r + rr, s] = av[r + rr, s] + bv[r + rr, s]

            out_pend[c] = pltpu.async_copy(
                av, out_hbm.at[pl.ds(base + c * CH, CH)], so[p])
        for c in sorted(out_pend):
            out_pend.pop(c).wait()

    return k(word_emb, comb_table, word_ids, comb_ids)


def kernel(input_ids, token_type_ids, position_ids, word_emb, pos_emb, type_emb):
    B, S = input_ids.shape
    comb_table = _build_combined(pos_emb, type_emb)
    wid = input_ids.astype(jnp.int32)
    cid = (token_type_ids.astype(jnp.int32) * MAX_POS
           + position_ids.astype(jnp.int32))
    out = _gather_sum(word_emb, comb_table, wid, cid)
    return out.reshape(B, S, EMBED)
